# baseline (speedup 1.0000x reference)
import functools

import jax
import jax.numpy as jnp
from jax import lax
from jax.experimental import pallas as pl
from jax.experimental.pallas import tpu as pltpu
from jax.experimental.pallas import tpu_sc as plsc

H = 4096
W = 4096
NBINS = 1024
NC = 2
NS = 16
NW = NC * NS
ROWS_PER_W = H // NW
R = 8
BLOCKS = ROWS_PER_W // R
L = 16
CHUNKS = W // L
BUFLEN = (R + 1) * W + L

_mesh = plsc.VectorSubcoreMesh(core_axis_name="c", subcore_axis_name="s")


def _to_bin(v):
    y = v * jnp.float32(NBINS - 1)
    ti = y.astype(jnp.int32)
    return jnp.where(ti.astype(jnp.float32) < y, ti + 1, ti)


@functools.partial(
    pl.kernel,
    out_type=jax.ShapeDtypeStruct((NW, NBINS), jnp.int32),
    mesh=_mesh,
    compiler_params=pltpu.CompilerParams(needs_layout_passes=False),
    scratch_types=[
        pltpu.VMEM((BUFLEN,), jnp.float32),
        pltpu.VMEM((BUFLEN,), jnp.float32),
        pltpu.VMEM((L * NBINS,), jnp.int32),
        pltpu.VMEM((NBINS,), jnp.int32),
        pltpu.SemaphoreType.DMA,
        pltpu.SemaphoreType.DMA,
    ],
)
def _hist_kernel(img_hbm, out_hbm, fbuf_a, fbuf_b, hist, obuf, sem_a, sem_b):
    wid = lax.axis_index("s") * NC + lax.axis_index("c")

    iota16 = lax.iota(jnp.int32, L)
    iota_u = lax.iota(jnp.uint32, L)
    ones = jnp.ones((L,), jnp.int32)
    mones = -ones
    zeros = jnp.zeros((L,), jnp.int32)
    mlast = iota16 < (L - 1)
    shift1 = jnp.minimum(iota16 + 1, L - 1)
    zero_idx = jnp.zeros((L,), jnp.int32)


    def dma_rows(bb, buf, sem):
        row0 = wid * ROWS_PER_W + bb * R
        copies = [
            pltpu.make_async_copy(img_hbm.at[row0 + r],
                                  buf.at[pl.ds(r * W, W)], sem)
            for r in range(R)
        ]
        halo = pltpu.make_async_copy(img_hbm.at[row0 + R],
                                     buf.at[pl.ds(R * W, W)], sem)
        return copies, halo, row0 + R < H

    def start_dma(bb, buf, sem):
        copies, halo, has_halo = dma_rows(bb, buf, sem)
        for c in copies:
            c.start()

        @pl.when(has_halo)
        def _():
            halo.start()

    def wait_dma(bb, buf, sem):
        copies, halo, has_halo = dma_rows(bb, buf, sem)
        for c in copies:
            c.wait()

        @pl.when(has_halo)
        def _():
            halo.wait()

    def process(bb, buf):
        row0 = wid * ROWS_PER_W + bb * R
        fast = row0 + R < H

        def conv_chunk(off):
            b = _to_bin(buf[pl.ds(off, L)])
            return plsc.bitcast((b << 4) | iota16, jnp.uint32)

        def scat(idx_u, val, mask=None):
            plsc.addupdate_scatter(hist, [plsc.bitcast(idx_u, jnp.int32)],
                                   val, mask=mask)

        def retag(idx_u):
            return (idx_u & jnp.uint32(0xFFFFFFF0)) | iota_u

        RH = R // 2

        def col_scatter(prev, sh, edge_mask):
            iy = [jnp.maximum(prev[i], sh[i]) for i in range(RH + 1)]
            for i in range(RH):
                ix = jnp.maximum(prev[i], prev[i + 1])
                isq = jnp.maximum(iy[i], iy[i + 1])
                scat(prev[i], ones)
                scat(ix, mones)
                scat(retag(iy[i]), mones, mask=edge_mask)
                scat(retag(isq), ones, mask=edge_mask)

        def walk(r0):
            def load_col(jb):
                return tuple(conv_chunk(jb + (r0 + i) * W)
                             for i in range(RH + 1))

            first = load_col(0)

            @plsc.parallel_loop(1, CHUNKS, carry=first)
            def chunkf(c, prev):
                cur = load_col(c * L)
                sh = [
                    jnp.where(
                        mlast,
                        prev[i].at[shift1].get(mode="promise_in_bounds"),
                        cur[i].at[zero_idx].get(mode="promise_in_bounds"),
                    )
                    for i in range(RH + 1)
                ]
                col_scatter(prev, sh, None)
                return cur

            last = chunkf
            sh = [last[i].at[shift1].get(mode="promise_in_bounds")
                  for i in range(RH + 1)]
            col_scatter(last, sh, mlast)

        @pl.when(fast)
        def _fast():
            walk(0)
            walk(RH)

        @pl.when(jnp.logical_not(fast))
        def _slow():
            @plsc.parallel_loop(0, (R * W) // L, unroll=4)
            def conv(t):
                off = t * L
                buf[pl.ds(off, L)] = plsc.bitcast(conv_chunk(off),
                                                  jnp.float32)

            def bins(off):
                return plsc.bitcast(buf[pl.ds(off, L)], jnp.uint32)

            def rowf(r, c0):
                gi = row0 + r

                @pl.when(gi < H - 1)
                def _full_row():
                    @plsc.parallel_loop(0, CHUNKS - 1, unroll=5)
                    def chunkf(c):
                        base = r * W + c * L
                        ia = bins(base)
                        iar = bins(base + 1)
                        iad = bins(base + W)
                        iadr = bins(base + W + 1)
                        ix = jnp.maximum(ia, iad)
                        iy = jnp.maximum(ia, iar)
                        isq = jnp.maximum(iy, jnp.maximum(iad, iadr))
                        scat(ia, ones)
                        scat(ix, mones)
                        scat(retag(iy), mones)
                        scat(retag(isq), ones)

                    base = r * W + (CHUNKS - 1) * L
                    ia = bins(base)
                    iar = bins(base + 1)
                    iad = bins(base + W)
                    iadr = bins(base + W + 1)
                    ix = jnp.maximum(ia, iad)
                    iy = jnp.maximum(ia, iar)
                    isq = jnp.maximum(iy, jnp.maximum(iad, iadr))
                    scat(ia, ones)
                    scat(ix, mones)
                    scat(retag(iy), mones, mask=mlast)
                    scat(retag(isq), ones, mask=mlast)

                @pl.when(gi == H - 1)
                def _last_row():
                    @plsc.parallel_loop(0, CHUNKS - 1, unroll=5)
                    def chunkv(c):
                        base = r * W + c * L
                        ia = bins(base)
                        iar = bins(base + 1)
                        scat(ia, ones)
                        scat(retag(jnp.maximum(ia, iar)), mones)

                    base = r * W + (CHUNKS - 1) * L
                    ia = bins(base)
                    iar = bins(base + 1)
                    scat(ia, ones)
                    scat(retag(jnp.maximum(ia, iar)), mones, mask=mlast)

                return c0

            lax.fori_loop(0, R, rowf, 0)

    start_dma(0, fbuf_a, sem_a)

    def zero_hist(i, carry):
        hist[pl.ds(i * L, L)] = zeros
        return carry

    lax.fori_loop(0, (L * NBINS) // L, zero_hist, 0)

    def outer(k, carry):
        b0 = 2 * k
        wait_dma(b0, fbuf_a, sem_a)
        start_dma(b0 + 1, fbuf_b, sem_b)
        plsc.subcore_barrier()
        process(b0, fbuf_a)
        wait_dma(b0 + 1, fbuf_b, sem_b)

        @pl.when(b0 + 2 < BLOCKS)
        def _():
            start_dma(b0 + 2, fbuf_a, sem_a)

        plsc.subcore_barrier()
        process(b0 + 1, fbuf_b)
        return carry

    lax.fori_loop(0, BLOCKS // 2, outer, 0)

    gidx = iota16 * L

    def fold(cidx, carry):
        acc = plsc.load_gather(hist, [gidx + cidx * (L * L)])
        for lane in range(1, L):
            acc = acc + plsc.load_gather(hist, [gidx + (cidx * (L * L) + lane)])
        obuf[pl.ds(cidx * L, L)] = acc
        return carry

    lax.fori_loop(0, NBINS // L, fold, 0)
    pltpu.sync_copy(obuf, out_hbm.at[wid])


@functools.partial(
    pl.kernel,
    out_type=jax.ShapeDtypeStruct((NBINS,), jnp.int32),
    mesh=_mesh,
    compiler_params=pltpu.CompilerParams(needs_layout_passes=False),
    scratch_types=[
        pltpu.VMEM((NW, NBINS), jnp.int32),
        pltpu.VMEM((NBINS,), jnp.int32),
    ],
)
def _finalize_kernel(part_hbm, out_hbm, pbuf, obuf):
    wid = lax.axis_index("s") * NC + lax.axis_index("c")

    onehot0 = (lax.iota(jnp.int32, L) == 0).astype(jnp.int32)
    fifteen = jnp.full((L,), L - 1, jnp.int32)

    @pl.when(wid == 0)
    def _():
        pltpu.sync_copy(part_hbm, pbuf)

        def chunk(cidx, carry_vec):
            off = cidx * L
            acc = pbuf[0, pl.ds(off, L)]
            for w in range(1, NW):
                acc = acc + pbuf[w, pl.ds(off, L)]
            acc = acc + carry_vec * onehot0
            cum = plsc.cumsum(acc)
            obuf[pl.ds(off, L)] = cum
            return cum.at[fifteen].get(mode="promise_in_bounds")

        lax.fori_loop(0, NBINS // L, chunk, jnp.zeros((L,), jnp.int32))
        pltpu.sync_copy(obuf, out_hbm)


def kernel(img_arr):
    part = _hist_kernel(img_arr)
    return _finalize_kernel(part)
